# baseline (speedup 1.0000x reference)
import jax
import jax.numpy as jnp
from jax.experimental import pallas as pl

_MARGIN = 1.0
_C = 32
_N = 1024
_DIM = 256
_EPS = 1e-6
_BLK = 1024
_XBLK = 128


def _dot(a, b, dims):
    return jax.lax.dot_general(a, b, (dims, ((), ())),
                               preferred_element_type=jnp.float32)


def _loss_kernel(x_ref, w_ref, b_ref, pos_ref, neg_ref):
    w = w_ref[...]
    b = jnp.transpose(b_ref[...])

    embT = _dot(w, x_ref[...], ((1,), (1,))) + b
    _loss_body(embT, pos_ref, neg_ref)


def _loss_body(embT, pos_ref, neg_ref):
    mxT = jnp.max(embT, axis=0, keepdims=True)
    riota = jax.lax.broadcasted_iota(jnp.int32, (_C, _N), 0).astype(
        jnp.float32)
    lblT = jnp.min(jnp.where(embT == mxT, riota, float(_C)),
                   axis=0, keepdims=True)
    onehotT = (riota == lblT).astype(jnp.float32)

    ones_1c = jnp.ones((1, _C), jnp.float32)
    ones_1n = jnp.ones((1, _N), jnp.float32)
    ones_n1 = jnp.ones((_N, 1), jnp.float32)

    w_col = _dot(onehotT, ones_n1, ((1,), (0,)))
    w_row = _dot(ones_1n, onehotT, ((1,), (1,)))
    w_safe = jnp.where(w_col == 0.0, 1.0, w_col)

    sums = _dot(onehotT, embT, ((1,), (1,)))
    means = sums / w_safe

    r_col = jnp.sum(means * means, axis=1, keepdims=True)
    s_col = jnp.sum(means, axis=1, keepdims=True)
    r_row = _dot(ones_1c, means * means, ((1,), (1,)))
    s_row = _dot(ones_1c, means, ((1,), (1,)))
    gm = _dot(means, means, ((1,), (1,)))
    d2 = r_col + r_row - 2.0 * gm + 2.0 * _EPS * (s_col - s_row) \
        + _C * _EPS * _EPS
    d2 = jnp.maximum(d2, 1e-12)
    ir = jax.lax.broadcasted_iota(jnp.int32, (_C, _C), 0)
    ic = jax.lax.broadcasted_iota(jnp.int32, (_C, _C), 1)
    valid = (w_col > 0.0) & (w_row > 0.0) & (ic > ir)
    min_d2 = jnp.min(jnp.where(valid, d2, 1e30), axis=(0, 1), keepdims=True)
    neg = jnp.maximum(_MARGIN - min_d2, 0.0)
    neg = neg * neg
    n_nonempty = jnp.sum((w_col > 0.0).astype(jnp.float32),
                         axis=(0, 1), keepdims=True)
    neg_ref[...] = jnp.where(n_nonempty > 1.0, neg, 0.0)

    expectsT = _dot(means, onehotT, ((0,), (0,)))
    e2T = embT - expectsT
    e2Tsq = e2T * e2T
    teT = (2.0 * _EPS) * e2T
    rcT = jnp.sum(e2Tsq + teT, axis=0, keepdims=True)
    rrT = jnp.sum(e2Tsq - teT, axis=0, keepdims=True) \
        + _C * _EPS * _EPS
    a_aug = jnp.concatenate([-2.0 * e2T, rcT, ones_1n], axis=0)
    b_aug = jnp.concatenate([e2T, ones_1n, rrT], axis=0)

    nb = _N // _BLK
    q_acc = [None] * nb
    for rb in range(nb):
        ra = slice(rb * _BLK, (rb + 1) * _BLK)
        for cb in range(rb, nb):
            ca = slice(cb * _BLK, (cb + 1) * _BLK)
            d2p = _dot(a_aug[:, ra], b_aug[:, ca], ((0,), (0,)))
            dist = jnp.sqrt(jnp.maximum(d2p, 1e-12))
            qa = _dot(onehotT[:, ca], dist, ((1,), (1,)))
            q_acc[rb] = qa if q_acc[rb] is None else q_acc[rb] + qa
            if cb != rb:
                qb = _dot(onehotT[:, ra], dist, ((1,), (0,)))
                q_acc[cb] = qb if q_acc[cb] is None else q_acc[cb] + qb
    qT = jnp.concatenate(q_acc, axis=1)
    pickedT = jnp.sum(onehotT * qT, axis=0, keepdims=True)

    w2 = w_row - 1.0
    inv_w3 = 1.0 / jnp.where(w2 <= 0.0, 1.0, w2)
    a_sT = _dot(inv_w3, onehotT, ((1,), (0,)))
    pos_ref[...] = jnp.sum(a_sT * pickedT, axis=(0, 1),
                           keepdims=True) / _C


def kernel(embeddings, W, b):
    pos, neg = pl.pallas_call(
        _loss_kernel,
        out_shape=(
            jax.ShapeDtypeStruct((1, 1), jnp.float32),
            jax.ShapeDtypeStruct((1, 1), jnp.float32),
        ),
    )(embeddings, W, b.reshape(1, _C))
    return pos[0, 0], neg[0, 0]


# speedup vs baseline: 1.0518x; 1.0518x over previous
import jax
import jax.numpy as jnp
from jax.experimental import pallas as pl

_MARGIN = 1.0
_C = 32
_N = 1024
_DIM = 256
_EPS = 1e-6
_BLK = 512
_XBLK = 128


def _dot(a, b, dims):
    return jax.lax.dot_general(a, b, (dims, ((), ())),
                               preferred_element_type=jnp.float32)


def _loss_kernel(x_ref, w_ref, b_ref, pos_ref, neg_ref):
    w = w_ref[...]
    b = jnp.transpose(b_ref[...])

    embT = _dot(w, x_ref[...], ((1,), (1,))) + b
    _loss_body(embT, pos_ref, neg_ref)


def _loss_body(embT, pos_ref, neg_ref):
    mxT = jnp.max(embT, axis=0, keepdims=True)
    riota = jax.lax.broadcasted_iota(jnp.int32, (_C, _N), 0).astype(
        jnp.float32)
    lblT = jnp.min(jnp.where(embT == mxT, riota, float(_C)),
                   axis=0, keepdims=True)
    onehotT = (riota == lblT).astype(jnp.float32)

    ones_1c = jnp.ones((1, _C), jnp.float32)
    ones_1n = jnp.ones((1, _N), jnp.float32)
    ones_n1 = jnp.ones((_N, 1), jnp.float32)

    w_col = _dot(onehotT, ones_n1, ((1,), (0,)))
    w_row = _dot(ones_1n, onehotT, ((1,), (1,)))
    w_safe = jnp.where(w_col == 0.0, 1.0, w_col)

    sums = _dot(onehotT, embT, ((1,), (1,)))
    means = sums / w_safe

    r_col = jnp.sum(means * means, axis=1, keepdims=True)
    s_col = jnp.sum(means, axis=1, keepdims=True)
    r_row = _dot(ones_1c, means * means, ((1,), (1,)))
    s_row = _dot(ones_1c, means, ((1,), (1,)))
    gm = _dot(means, means, ((1,), (1,)))
    d2 = r_col + r_row - 2.0 * gm + 2.0 * _EPS * (s_col - s_row) \
        + _C * _EPS * _EPS
    d2 = jnp.maximum(d2, 1e-12)
    ir = jax.lax.broadcasted_iota(jnp.int32, (_C, _C), 0)
    ic = jax.lax.broadcasted_iota(jnp.int32, (_C, _C), 1)
    valid = (w_col > 0.0) & (w_row > 0.0) & (ic > ir)
    min_d2 = jnp.min(jnp.where(valid, d2, 1e30), axis=(0, 1), keepdims=True)
    neg = jnp.maximum(_MARGIN - min_d2, 0.0)
    neg = neg * neg
    n_nonempty = jnp.sum((w_col > 0.0).astype(jnp.float32),
                         axis=(0, 1), keepdims=True)
    neg_ref[...] = jnp.where(n_nonempty > 1.0, neg, 0.0)

    expectsT = _dot(means, onehotT, ((0,), (0,)))
    e2T = embT - expectsT
    e2Tsq = e2T * e2T
    teT = (2.0 * _EPS) * e2T
    rcT = jnp.sum(e2Tsq + teT, axis=0, keepdims=True)
    rrT = jnp.sum(e2Tsq - teT, axis=0, keepdims=True) \
        + _C * _EPS * _EPS
    a_aug = jnp.concatenate([-2.0 * e2T, rcT, ones_1n], axis=0)
    b_aug = jnp.concatenate([e2T, ones_1n, rrT], axis=0)

    oh_bf = onehotT.astype(jnp.bfloat16)
    nb = _N // _BLK
    q_acc = [None] * nb
    for rb in range(nb):
        ra = slice(rb * _BLK, (rb + 1) * _BLK)
        for cb in range(rb, nb):
            ca = slice(cb * _BLK, (cb + 1) * _BLK)
            d2p = _dot(a_aug[:, ra], b_aug[:, ca], ((0,), (0,)))
            dist = jnp.sqrt(jnp.maximum(d2p, 1e-12)).astype(jnp.bfloat16)
            qa = _dot(oh_bf[:, ca], dist, ((1,), (1,)))
            q_acc[rb] = qa if q_acc[rb] is None else q_acc[rb] + qa
            if cb != rb:
                qb = _dot(oh_bf[:, ra], dist, ((1,), (0,)))
                q_acc[cb] = qb if q_acc[cb] is None else q_acc[cb] + qb
    qT = jnp.concatenate(q_acc, axis=1)
    pickedT = jnp.sum(onehotT * qT, axis=0, keepdims=True)

    w2 = w_row - 1.0
    inv_w3 = 1.0 / jnp.where(w2 <= 0.0, 1.0, w2)
    a_sT = _dot(inv_w3, onehotT, ((1,), (0,)))
    pos_ref[...] = jnp.sum(a_sT * pickedT, axis=(0, 1),
                           keepdims=True) / _C


def kernel(embeddings, W, b):
    pos, neg = pl.pallas_call(
        _loss_kernel,
        out_shape=(
            jax.ShapeDtypeStruct((1, 1), jnp.float32),
            jax.ShapeDtypeStruct((1, 1), jnp.float32),
        ),
    )(embeddings, W, b.reshape(1, _C))
    return pos[0, 0], neg[0, 0]
